# baseline (speedup 1.0000x reference)
import functools

import numpy as np
import jax
import jax.numpy as jnp
from jax import lax
from jax.experimental import pallas as pl
from jax.experimental.pallas import tpu as pltpu
from jax.experimental.pallas import tpu_sc as plsc

_INPUT_SIZE = 384
_N_FRAMES = 256
_N_LM = 227
_ROW = 543
_FRAME = 543 * 543
_COL_B = 6
_SEG = (0, 160, 181, 206, 227)
_KPF = 2 * _N_LM
_KPAD = 464
_L = 16
_NCHUNK = _KPAD // _L
_GCHUNK = 128

_NC = 2
_NS = 16
_NW = _NC * _NS
_FPT = _N_FRAMES // _NW
_KPT = _FPT * _KPAD
_NGC = _KPT // _GCHUNK
_PPT = (_INPUT_SIZE - _N_FRAMES) // _NW

_RUNS = tuple((2 * _SEG[s], 2 * _SEG[s + 1]) for s in range(4))


def _chunk_pieces(ch):
    pieces = []
    for s, (lo, hi) in enumerate(_RUNS):
        a = max(lo, _L * ch) - _L * ch
        b = min(hi, _L * ch + _L) - _L * ch
        if a < b:
            pieces.append((s, a, b))
    return pieces


def _build_offsets():
    f = np.arange(_N_FRAMES, dtype=np.int64)[:, None]
    k = np.arange(_KPAD, dtype=np.int64)[None, :]
    r = np.minimum(k, _KPF - 1) // 2
    c = np.where(k < _KPF, k % 2, 0)
    o = f * _FRAME + r * _ROW + _COL_B * c
    o = np.where(k < _KPF, o, 0)
    return o.astype(np.int32).reshape(_NW, _NGC, _GCHUNK)


_OFFS_NP = _build_offsets()


def _rsqrt_newton(s):
    i = lax.bitcast_convert_type(s, jnp.int32)
    y = lax.bitcast_convert_type(np.int32(0x5F3759DF) - (i >> 1), jnp.float32)
    for _ in range(3):
        y = y * (1.5 - 0.5 * s * y * y)
    return jnp.where(s == 0.0, 1.0, y)


@functools.cache
def _make_preprocess_sc():
    return pl.kernel(
        _preprocess_sc_body,
        out_type=jax.ShapeDtypeStruct((_INPUT_SIZE, _KPAD), jnp.float32),
        mesh=plsc.VectorSubcoreMesh(core_axis_name="c", subcore_axis_name="s",
                                    num_cores=_NC, num_subcores=_NS),
        scratch_types=[
            pltpu.VMEM((_NGC, _GCHUNK), jnp.int32),
            pltpu.VMEM((_KPT,), jnp.float32),
            pltpu.VMEM((_KPAD,), jnp.float32),
            pltpu.SemaphoreType.DMA,
        ],
        compiler_params=pltpu.CompilerParams(needs_layout_passes=False,
                                             use_tc_tiling_on_sc=False),
    )


def _preprocess_sc_body(data_hbm, offs_hbm, out_hbm, oidx_v, elems_v, pad_v, sem):
    wid = lax.axis_index("s") * _NC + lax.axis_index("c")

    pltpu.sync_copy(offs_hbm.at[wid], oidx_v)

    copies = []
    for j in range(_NGC):
        copies.append(
            pltpu.async_copy(
                data_hbm.at[oidx_v.at[j]],
                elems_v.at[pl.ds(j * _GCHUNK, _GCHUNK)],
                sem,
            )
        )
    for cp in copies:
        cp.wait()

    lane_iota = lax.iota(jnp.int32, 16)
    even = (lane_iota & 1) == 0
    zeros = jnp.zeros((16,), jnp.float32)

    def frame_body(fl, _):
        base = fl * _KPAD
        accs = [zeros, zeros, zeros, zeros]
        for ch in range(_NCHUNK):
            v = elems_v[pl.ds(base + _L * ch, _L)]
            sq = v * v
            for s, a, b in _chunk_pieces(ch):
                if a == 0 and b == _L:
                    accs[s] = accs[s] + sq
                else:
                    m = (lane_iota >= a) & (lane_iota < b)
                    accs[s] = accs[s] + jnp.where(m, sq, 0.0)

        rvecs = []
        for s in range(4):
            se = jnp.sum(jnp.where(even, accs[s], 0.0))
            so = jnp.sum(jnp.where(even, 0.0, accs[s]))
            rvecs.append(jnp.where(even, _rsqrt_newton(se), _rsqrt_newton(so)))

        for ch in range(_NCHUNK):
            pieces = _chunk_pieces(ch)
            rv = rvecs[pieces[-1][0]]
            for s, a, b in reversed(pieces[:-1]):
                rv = jnp.where(lane_iota < b, rvecs[s], rv)
            v = elems_v[pl.ds(base + _L * ch, _L)]
            y = v * rv
            y = jnp.where(y != y, 0.0, y)
            elems_v[pl.ds(base + _L * ch, _L)] = y

        pltpu.sync_copy(elems_v.at[pl.ds(base, _KPAD)],
                        out_hbm.at[wid * _FPT + fl])
        return ()

    lax.fori_loop(0, _FPT, frame_body, (), unroll=False)

    neg1 = jnp.full((16,), -1.0, jnp.float32)
    for ch in range(_NCHUNK):
        pad_v[pl.ds(_L * ch, _L)] = neg1
    for j in range(_PPT):
        pltpu.sync_copy(pad_v, out_hbm.at[_N_FRAMES + wid * _PPT + j])


def kernel(data):
    n_frames = data.shape[0]
    flat = data.reshape(-1)
    padded = _make_preprocess_sc()(flat, jnp.asarray(_OFFS_NP))
    out = padded[:, :_KPF]
    idxs = jnp.concatenate(
        [jnp.arange(n_frames, dtype=jnp.int32),
         jnp.full((_INPUT_SIZE - n_frames,), -1, dtype=jnp.int32)]
    )
    return (out, idxs)


# speedup vs baseline: 10.4471x; 10.4471x over previous
import functools

import numpy as np
import jax
import jax.numpy as jnp
from jax import lax
from jax.experimental import pallas as pl
from jax.experimental.pallas import tpu as pltpu
from jax.experimental.pallas import tpu_sc as plsc

_INPUT_SIZE = 384
_N_FRAMES = 256
_N_LM = 227
_COL_B = 6
_SEG = (0, 160, 181, 206, 227)
_KPF = 2 * _N_LM
_KPAD = 464
_L = 16
_NCHUNK = _KPAD // _L
_RSLAB = 232
_CSLAB = 128

_NC = 2
_NS = 16
_NW = _NC * _NS
_FPT = _N_FRAMES // _NW

_RUNS = tuple((2 * _SEG[s], 2 * _SEG[s + 1]) for s in range(4))


def _chunk_pieces(ch):
    pieces = []
    for s, (lo, hi) in enumerate(_RUNS):
        a = max(lo, _L * ch) - _L * ch
        b = min(hi, _L * ch + _L) - _L * ch
        if a < b:
            pieces.append((s, a, b))
    return pieces


def _rsqrt_newton(s):
    i = lax.bitcast_convert_type(s, jnp.int32)
    y = lax.bitcast_convert_type(np.int32(0x5F3759DF) - (i >> 1), jnp.float32)
    for _ in range(3):
        y = y * (1.5 - 0.5 * s * y * y)
    return jnp.where(s == 0.0, 1.0, y)


@functools.cache
def _make_preprocess_sc():
    return pl.kernel(
        _preprocess_sc_body,
        out_type=jax.ShapeDtypeStruct((_INPUT_SIZE, _KPAD), jnp.float32),
        mesh=plsc.VectorSubcoreMesh(core_axis_name="c", subcore_axis_name="s",
                                    num_cores=_NC, num_subcores=_NS),
        scratch_types=[
            pltpu.VMEM((_RSLAB, _CSLAB), jnp.float32),
            pltpu.VMEM((_FPT, _KPAD), jnp.float32),
            pltpu.SemaphoreType.DMA,
        ],
        compiler_params=pltpu.CompilerParams(needs_layout_passes=False),
    )


def _preprocess_sc_body(data_hbm, out_hbm, fbuf_v, vals_v, sem):
    wid = lax.axis_index("s") * _NC + lax.axis_index("c")

    lane_iota = lax.iota(jnp.int32, 16)
    even = (lane_iota & 1) == 0
    zeros = jnp.zeros((16,), jnp.float32)

    def frame_body(fl, _):
        pltpu.sync_copy(
            data_hbm.at[wid * _FPT + fl, pl.ds(0, _RSLAB), pl.ds(0, _CSLAB)],
            fbuf_v,
        )

        chunk_vals = []
        accs = [zeros, zeros, zeros, zeros]
        for ch in range(_NCHUNK):
            kvec = _L * ch + lane_iota
            rvec = kvec >> 1
            cvec = (kvec & 1) * _COL_B
            v = plsc.load_gather(fbuf_v, [rvec, cvec])
            chunk_vals.append(v)
            sq = v * v
            for s, a, b in _chunk_pieces(ch):
                if a == 0 and b == _L:
                    accs[s] = accs[s] + sq
                else:
                    m = (lane_iota >= a) & (lane_iota < b)
                    accs[s] = accs[s] + jnp.where(m, sq, 0.0)

        rvecs = []
        for s in range(4):
            se = jnp.sum(jnp.where(even, accs[s], 0.0))
            so = jnp.sum(jnp.where(even, 0.0, accs[s]))
            rvecs.append(jnp.where(even, _rsqrt_newton(se), _rsqrt_newton(so)))

        for ch in range(_NCHUNK):
            pieces = _chunk_pieces(ch)
            rv = rvecs[pieces[-1][0]]
            for s, a, b in reversed(pieces[:-1]):
                rv = jnp.where(lane_iota < b, rvecs[s], rv)
            y = chunk_vals[ch] * rv
            y = jnp.where(y != y, 0.0, y)
            vals_v[fl, pl.ds(_L * ch, _L)] = y
        return ()

    lax.fori_loop(0, _FPT, frame_body, (), unroll=False)

    pltpu.sync_copy(vals_v, out_hbm.at[pl.ds(wid * _FPT, _FPT)])

    @pl.when(wid < (_INPUT_SIZE - _N_FRAMES) // _FPT)
    def _():
        neg1 = jnp.full((16,), -1.0, jnp.float32)
        for r in range(_FPT):
            for ch in range(_NCHUNK):
                vals_v[r, pl.ds(_L * ch, _L)] = neg1
        pltpu.sync_copy(vals_v,
                        out_hbm.at[pl.ds(_N_FRAMES + wid * _FPT, _FPT)])


def kernel(data):
    n_frames = data.shape[0]
    padded = _make_preprocess_sc()(data)
    out = padded[:, :_KPF]
    idxs = jnp.concatenate(
        [jnp.arange(n_frames, dtype=jnp.int32),
         jnp.full((_INPUT_SIZE - n_frames,), -1, dtype=jnp.int32)]
    )
    return (out, idxs)


# speedup vs baseline: 10.4570x; 1.0009x over previous
import functools

import numpy as np
import jax
import jax.numpy as jnp
from jax import lax
from jax.experimental import pallas as pl
from jax.experimental.pallas import tpu as pltpu
from jax.experimental.pallas import tpu_sc as plsc

_INPUT_SIZE = 384
_N_FRAMES = 256
_N_LM = 227
_COL_B = 6
_SEG = (0, 160, 181, 206, 227)
_KPF = 2 * _N_LM
_KPAD = 464
_L = 16
_NCHUNK = _KPAD // _L
_RSLAB = 232
_CSLAB = 128
_TROWS = _RSLAB // 8

_NC = 2
_NS = 16
_NW = _NC * _NS
_FPT = _N_FRAMES // _NW

_RUNS = tuple((2 * _SEG[s], 2 * _SEG[s + 1]) for s in range(4))


def _chunk_pieces(ch):
    pieces = []
    for s, (lo, hi) in enumerate(_RUNS):
        a = max(lo, _L * ch) - _L * ch
        b = min(hi, _L * ch + _L) - _L * ch
        if a < b:
            pieces.append((s, a, b))
    return pieces


def _rsqrt_newton(s):
    i = lax.bitcast_convert_type(s, jnp.int32)
    y = lax.bitcast_convert_type(np.int32(0x5F3759DF) - (i >> 1), jnp.float32)
    for _ in range(3):
        y = y * (1.5 - 0.5 * s * y * y)
    return jnp.where(s == 0.0, 1.0, y)


@functools.cache
def _make_preprocess_sc():
    return pl.kernel(
        _preprocess_sc_body,
        out_type=jax.ShapeDtypeStruct((_INPUT_SIZE, _KPAD), jnp.float32),
        mesh=plsc.VectorSubcoreMesh(core_axis_name="c", subcore_axis_name="s",
                                    num_cores=_NC, num_subcores=_NS),
        scratch_types=[
            pltpu.VMEM((_RSLAB, _CSLAB), jnp.float32),
            pltpu.VMEM((_RSLAB, _CSLAB), jnp.float32),
            pltpu.VMEM((_FPT, _KPAD), jnp.float32),
            pltpu.SemaphoreType.DMA,
            pltpu.SemaphoreType.DMA,
        ],
        compiler_params=pltpu.CompilerParams(needs_layout_passes=False),
    )


def _preprocess_sc_body(data_hbm, out_hbm, fbuf0_v, fbuf1_v, vals_v,
                        sem0, sem1):
    wid = lax.axis_index("s") * _NC + lax.axis_index("c")
    bufs = (fbuf0_v, fbuf1_v)
    sems = (sem0, sem1)

    lane_iota = lax.iota(jnp.int32, 16)
    even = (lane_iota & 1) == 0
    zeros = jnp.zeros((16,), jnp.float32)

    def fire(fl):
        b = fl & 1
        return [
            pltpu.async_copy(
                data_hbm.at[wid * _FPT + fl,
                            pl.ds(8 * tr, 8), pl.ds(0, _CSLAB)],
                bufs[b].at[pl.ds(8 * tr, 8)],
                sems[b],
            )
            for tr in range(_TROWS)
        ]

    def compute(fl):
        fbuf = bufs[fl & 1]
        chunk_vals = []
        accs = [zeros, zeros, zeros, zeros]
        for ch in range(_NCHUNK):
            kvec = _L * ch + lane_iota
            rvec = kvec >> 1
            cvec = (kvec & 1) * _COL_B
            v = plsc.load_gather(fbuf, [rvec, cvec])
            chunk_vals.append(v)
            sq = v * v
            for s, a, b in _chunk_pieces(ch):
                if a == 0 and b == _L:
                    accs[s] = accs[s] + sq
                else:
                    m = (lane_iota >= a) & (lane_iota < b)
                    accs[s] = accs[s] + jnp.where(m, sq, 0.0)

        rvecs = []
        for s in range(4):
            se = jnp.sum(jnp.where(even, accs[s], 0.0))
            so = jnp.sum(jnp.where(even, 0.0, accs[s]))
            rvecs.append(jnp.where(even, _rsqrt_newton(se), _rsqrt_newton(so)))

        for ch in range(_NCHUNK):
            pieces = _chunk_pieces(ch)
            rv = rvecs[pieces[-1][0]]
            for s, a, b in reversed(pieces[:-1]):
                rv = jnp.where(lane_iota < b, rvecs[s], rv)
            y = chunk_vals[ch] * rv
            y = jnp.where(y != y, 0.0, y)
            vals_v[fl, pl.ds(_L * ch, _L)] = y

    pending = fire(0)
    for fl in range(_FPT):
        nxt = fire(fl + 1) if fl + 1 < _FPT else None
        for cp in pending:
            cp.wait()
        compute(fl)
        pending = nxt

    pltpu.sync_copy(vals_v, out_hbm.at[pl.ds(wid * _FPT, _FPT)])

    @pl.when(wid < (_INPUT_SIZE - _N_FRAMES) // _FPT)
    def _():
        neg1 = jnp.full((16,), -1.0, jnp.float32)
        for r in range(_FPT):
            for ch in range(_NCHUNK):
                vals_v[r, pl.ds(_L * ch, _L)] = neg1
        pltpu.sync_copy(vals_v,
                        out_hbm.at[pl.ds(_N_FRAMES + wid * _FPT, _FPT)])


def kernel(data):
    n_frames = data.shape[0]
    padded = _make_preprocess_sc()(data)
    out = padded[:, :_KPF]
    idxs = jnp.concatenate(
        [jnp.arange(n_frames, dtype=jnp.int32),
         jnp.full((_INPUT_SIZE - n_frames,), -1, dtype=jnp.int32)]
    )
    return (out, idxs)
